# Initial kernel scaffold; baseline (speedup 1.0000x reference)
import jax
import jax.numpy as jnp
from jax.experimental import pallas as pl


def kernel(x, A_tilde, s1_sct, s2_sct, s3_sct, W0, a0, W1, a1, Wg, bg):
    raise NotImplementedError("write your pallas kernel here")


# speedup vs baseline: 1.7713x; 1.7713x over previous
import jax
import jax.numpy as jnp
from jax.experimental import pallas as pl
from jax.experimental.pallas import tpu as pltpu

_HID = 32
_SMOO = 0.5
_BLK_ATT = 80
_BLK_FIN = 400


def _h_kernel(x_ref, w_ref, h_ref):
    h_ref[...] = jnp.dot(x_ref[...], w_ref[...],
                         preferred_element_type=jnp.float32)


def _att_kernel(a_ref, s1_ref, s2_ref, s3_ref, h_ref, a0_ref, a1_ref,
                wg_ref, sup_ref):
    i = pl.program_id(0)
    H = h_ref[...]
    c0 = jnp.dot(a_ref[...], H, preferred_element_type=jnp.float32)
    c1 = jnp.abs(jnp.dot(s1_ref[...], H, preferred_element_type=jnp.float32))
    c2 = jnp.abs(jnp.dot(s2_ref[...], H, preferred_element_type=jnp.float32))
    c3 = jnp.abs(jnp.dot(s3_ref[...], H, preferred_element_type=jnp.float32))
    hb = h_ref[pl.ds(i * _BLK_ATT, _BLK_ATT), :]

    heads = []
    for p, ap_ref in ((0, a0_ref), (1, a1_ref)):
        sl = slice(p * _HID, (p + 1) * _HID)
        ap = ap_ref[...]
        a_top, a_bot = ap[:_HID, :], ap[_HID:, :]
        e_h = jnp.dot(hb[:, sl], a_top, preferred_element_type=jnp.float32)
        es, cs = [], []
        for c in (c0, c1, c2, c3):
            cp = c[:, sl]
            cs.append(cp)
            e = e_h + jnp.dot(cp, a_bot, preferred_element_type=jnp.float32)
            es.append(jnp.where(e >= 0, e, 0.2 * e))
        e_all = jnp.concatenate(es, axis=1)
        m = jnp.max(e_all, axis=1, keepdims=True)
        w = jnp.exp(e_all - m)
        w = w / jnp.sum(w, axis=1, keepdims=True)
        acc = (w[:, 0:1] * cs[0] + w[:, 1:2] * cs[1]
               + w[:, 2:3] * cs[2] + w[:, 3:4] * cs[3])
        heads.append(jnp.maximum(acc, 0.0))

    h_cat = jnp.concatenate(heads, axis=1)
    sup_ref[...] = jnp.dot(h_cat, wg_ref[...],
                           preferred_element_type=jnp.float32)


def _final_kernel(a_ref, sup_ref, bg_ref, out_ref):
    i = pl.program_id(0)
    sup = sup_ref[...]
    conv = jnp.dot(a_ref[...], sup, preferred_element_type=jnp.float32)
    sb = sup_ref[pl.ds(i * _BLK_FIN, _BLK_FIN), :]
    o = (conv + _SMOO * sb) / (1.0 + _SMOO) + bg_ref[...]
    m = jnp.max(o, axis=1, keepdims=True)
    z = o - m
    lse = jnp.log(jnp.sum(jnp.exp(z), axis=1, keepdims=True))
    out_ref[...] = z - lse


def kernel(x, A_tilde, s1_sct, s2_sct, s3_sct, W0, a0, W1, a1, Wg, bg):
    n, nfeat = x.shape
    hid2 = 2 * _HID
    nclass = Wg.shape[1]
    wcat = jnp.concatenate([W0, W1], axis=1)
    bg2 = bg.reshape(1, nclass)

    H = pl.pallas_call(
        _h_kernel,
        out_shape=jax.ShapeDtypeStruct((n, hid2), jnp.float32),
        in_specs=[pl.BlockSpec((n, nfeat), lambda: (0, 0)),
                  pl.BlockSpec((nfeat, hid2), lambda: (0, 0))],
        out_specs=pl.BlockSpec((n, hid2), lambda: (0, 0)),
    )(x, wcat)

    grid_att = n // _BLK_ATT
    support = pl.pallas_call(
        _att_kernel,
        grid=(grid_att,),
        out_shape=jax.ShapeDtypeStruct((n, nclass), jnp.float32),
        in_specs=[
            pl.BlockSpec((_BLK_ATT, n), lambda i: (i, 0)),
            pl.BlockSpec((_BLK_ATT, n), lambda i: (i, 0)),
            pl.BlockSpec((_BLK_ATT, n), lambda i: (i, 0)),
            pl.BlockSpec((_BLK_ATT, n), lambda i: (i, 0)),
            pl.BlockSpec((n, hid2), lambda i: (0, 0)),
            pl.BlockSpec((hid2, 1), lambda i: (0, 0)),
            pl.BlockSpec((hid2, 1), lambda i: (0, 0)),
            pl.BlockSpec((hid2, nclass), lambda i: (0, 0)),
        ],
        out_specs=pl.BlockSpec((_BLK_ATT, nclass), lambda i: (i, 0)),
        compiler_params=pltpu.CompilerParams(
            dimension_semantics=("arbitrary",),
        ),
    )(A_tilde, s1_sct, s2_sct, s3_sct, H, a0, a1, Wg)

    grid_fin = n // _BLK_FIN
    out = pl.pallas_call(
        _final_kernel,
        grid=(grid_fin,),
        out_shape=jax.ShapeDtypeStruct((n, nclass), jnp.float32),
        in_specs=[
            pl.BlockSpec((_BLK_FIN, n), lambda i: (i, 0)),
            pl.BlockSpec((n, nclass), lambda i: (0, 0)),
            pl.BlockSpec((1, nclass), lambda i: (0, 0)),
        ],
        out_specs=pl.BlockSpec((_BLK_FIN, nclass), lambda i: (i, 0)),
        compiler_params=pltpu.CompilerParams(
            dimension_semantics=("arbitrary",),
        ),
    )(A_tilde, support, bg2)
    return out


# speedup vs baseline: 1.7720x; 1.0004x over previous
import jax
import jax.numpy as jnp
from jax.experimental import pallas as pl
from jax.experimental.pallas import tpu as pltpu

_HID = 32
_SMOO = 0.5
_BLK_ATT = 80
_BLK_FIN = 400


def _h_kernel(x_ref, w_ref, h_ref):
    h_ref[...] = jnp.dot(x_ref[...], w_ref[...],
                         preferred_element_type=jnp.float32)


def _att_kernel(a_ref, s1_ref, s2_ref, s3_ref, h_ref, a0_ref, a1_ref,
                wg_ref, sup_ref):
    i = pl.program_id(0)
    H = h_ref[...]
    c0 = jnp.dot(a_ref[...], H, preferred_element_type=jnp.float32)
    c1 = jnp.abs(jnp.dot(s1_ref[...], H, preferred_element_type=jnp.float32))
    c2 = jnp.abs(jnp.dot(s2_ref[...], H, preferred_element_type=jnp.float32))
    c3 = jnp.abs(jnp.dot(s3_ref[...], H, preferred_element_type=jnp.float32))
    hb = h_ref[pl.ds(i * _BLK_ATT, _BLK_ATT), :]

    heads = []
    for p, ap_ref in ((0, a0_ref), (1, a1_ref)):
        sl = slice(p * _HID, (p + 1) * _HID)
        ap = ap_ref[...]
        a_top, a_bot = ap[:_HID, :], ap[_HID:, :]
        e_h = jnp.dot(hb[:, sl], a_top, preferred_element_type=jnp.float32)
        es, cs = [], []
        for c in (c0, c1, c2, c3):
            cp = c[:, sl]
            cs.append(cp)
            e = e_h + jnp.dot(cp, a_bot, preferred_element_type=jnp.float32)
            es.append(jnp.where(e >= 0, e, 0.2 * e))
        e_all = jnp.concatenate(es, axis=1)
        m = jnp.max(e_all, axis=1, keepdims=True)
        w = jnp.exp(e_all - m)
        w = w / jnp.sum(w, axis=1, keepdims=True)
        acc = (w[:, 0:1] * cs[0] + w[:, 1:2] * cs[1]
               + w[:, 2:3] * cs[2] + w[:, 3:4] * cs[3])
        heads.append(jnp.maximum(acc, 0.0))

    h_cat = jnp.concatenate(heads, axis=1)
    sup_ref[...] = jnp.dot(h_cat, wg_ref[...],
                           preferred_element_type=jnp.float32)


def _final_kernel(a_ref, sup_ref, bg_ref, out_ref):
    i = pl.program_id(0)
    sup = sup_ref[...]
    conv = jnp.dot(a_ref[...], sup, preferred_element_type=jnp.float32)
    sb = sup_ref[pl.ds(i * _BLK_FIN, _BLK_FIN), :]
    o = (conv + _SMOO * sb) / (1.0 + _SMOO) + bg_ref[...]
    m = jnp.max(o, axis=1, keepdims=True)
    z = o - m
    lse = jnp.log(jnp.sum(jnp.exp(z), axis=1, keepdims=True))
    out_ref[...] = z - lse


def kernel(x, A_tilde, s1_sct, s2_sct, s3_sct, W0, a0, W1, a1, Wg, bg):
    n, nfeat = x.shape
    hid2 = 2 * _HID
    nclass = Wg.shape[1]
    wcat = jnp.concatenate([W0, W1], axis=1)
    bg2 = bg.reshape(1, nclass)

    H = pl.pallas_call(
        _h_kernel,
        out_shape=jax.ShapeDtypeStruct((n, hid2), jnp.float32),
        in_specs=[pl.BlockSpec((n, nfeat), lambda: (0, 0)),
                  pl.BlockSpec((nfeat, hid2), lambda: (0, 0))],
        out_specs=pl.BlockSpec((n, hid2), lambda: (0, 0)),
    )(x, wcat)

    grid_att = n // _BLK_ATT
    support = pl.pallas_call(
        _att_kernel,
        grid=(grid_att,),
        out_shape=jax.ShapeDtypeStruct((n, nclass), jnp.float32),
        in_specs=[
            pl.BlockSpec((_BLK_ATT, n), lambda i: (i, 0)),
            pl.BlockSpec((_BLK_ATT, n), lambda i: (i, 0)),
            pl.BlockSpec((_BLK_ATT, n), lambda i: (i, 0)),
            pl.BlockSpec((_BLK_ATT, n), lambda i: (i, 0)),
            pl.BlockSpec((n, hid2), lambda i: (0, 0)),
            pl.BlockSpec((hid2, 1), lambda i: (0, 0)),
            pl.BlockSpec((hid2, 1), lambda i: (0, 0)),
            pl.BlockSpec((hid2, nclass), lambda i: (0, 0)),
        ],
        out_specs=pl.BlockSpec((_BLK_ATT, nclass), lambda i: (i, 0)),
        compiler_params=pltpu.CompilerParams(
            dimension_semantics=("parallel",),
        ),
    )(A_tilde, s1_sct, s2_sct, s3_sct, H, a0, a1, Wg)

    grid_fin = n // _BLK_FIN
    out = pl.pallas_call(
        _final_kernel,
        grid=(grid_fin,),
        out_shape=jax.ShapeDtypeStruct((n, nclass), jnp.float32),
        in_specs=[
            pl.BlockSpec((_BLK_FIN, n), lambda i: (i, 0)),
            pl.BlockSpec((n, nclass), lambda i: (0, 0)),
            pl.BlockSpec((1, nclass), lambda i: (0, 0)),
        ],
        out_specs=pl.BlockSpec((_BLK_FIN, nclass), lambda i: (i, 0)),
        compiler_params=pltpu.CompilerParams(
            dimension_semantics=("parallel",),
        ),
    )(A_tilde, support, bg2)
    return out


# speedup vs baseline: 1.7746x; 1.0015x over previous
import jax
import jax.numpy as jnp
from jax.experimental import pallas as pl
from jax.experimental.pallas import tpu as pltpu

_HID = 32
_SMOO = 0.5
_BLK_ATT = 160
_BLK_FIN = 400


def _h_kernel(x_ref, w_ref, h_ref):
    h_ref[...] = jnp.dot(x_ref[...], w_ref[...],
                         preferred_element_type=jnp.float32)


def _att_kernel(a_ref, s1_ref, s2_ref, s3_ref, h_ref, hb_ref, a0_ref, a1_ref,
                wg_ref, sup_ref):
    H = h_ref[...]
    c0 = jnp.dot(a_ref[...], H, preferred_element_type=jnp.float32)
    c1 = jnp.abs(jnp.dot(s1_ref[...], H, preferred_element_type=jnp.float32))
    c2 = jnp.abs(jnp.dot(s2_ref[...], H, preferred_element_type=jnp.float32))
    c3 = jnp.abs(jnp.dot(s3_ref[...], H, preferred_element_type=jnp.float32))
    hb = hb_ref[...]

    heads = []
    for p, ap_ref in ((0, a0_ref), (1, a1_ref)):
        sl = slice(p * _HID, (p + 1) * _HID)
        ap = ap_ref[...]
        a_top, a_bot = ap[:_HID, :], ap[_HID:, :]
        e_h = jnp.dot(hb[:, sl], a_top, preferred_element_type=jnp.float32)
        es, cs = [], []
        for c in (c0, c1, c2, c3):
            cp = c[:, sl]
            cs.append(cp)
            e = e_h + jnp.dot(cp, a_bot, preferred_element_type=jnp.float32)
            es.append(jnp.where(e >= 0, e, 0.2 * e))
        e_all = jnp.concatenate(es, axis=1)
        m = jnp.max(e_all, axis=1, keepdims=True)
        w = jnp.exp(e_all - m)
        w = w / jnp.sum(w, axis=1, keepdims=True)
        acc = (w[:, 0:1] * cs[0] + w[:, 1:2] * cs[1]
               + w[:, 2:3] * cs[2] + w[:, 3:4] * cs[3])
        heads.append(jnp.maximum(acc, 0.0))

    h_cat = jnp.concatenate(heads, axis=1)
    sup_ref[...] = jnp.dot(h_cat, wg_ref[...],
                           preferred_element_type=jnp.float32)


def _final_kernel(a_ref, sup_ref, bg_ref, out_ref):
    i = pl.program_id(0)
    sup = sup_ref[...]
    conv = jnp.dot(a_ref[...], sup, preferred_element_type=jnp.float32)
    sb = sup_ref[pl.ds(i * _BLK_FIN, _BLK_FIN), :]
    o = (conv + _SMOO * sb) / (1.0 + _SMOO) + bg_ref[...]
    m = jnp.max(o, axis=1, keepdims=True)
    z = o - m
    lse = jnp.log(jnp.sum(jnp.exp(z), axis=1, keepdims=True))
    out_ref[...] = z - lse


def kernel(x, A_tilde, s1_sct, s2_sct, s3_sct, W0, a0, W1, a1, Wg, bg):
    n, nfeat = x.shape
    hid2 = 2 * _HID
    nclass = Wg.shape[1]
    wcat = jnp.concatenate([W0, W1], axis=1)
    bg2 = bg.reshape(1, nclass)

    H = pl.pallas_call(
        _h_kernel,
        out_shape=jax.ShapeDtypeStruct((n, hid2), jnp.float32),
        in_specs=[pl.BlockSpec((n, nfeat), lambda: (0, 0)),
                  pl.BlockSpec((nfeat, hid2), lambda: (0, 0))],
        out_specs=pl.BlockSpec((n, hid2), lambda: (0, 0)),
    )(x, wcat)

    grid_att = (n + _BLK_ATT - 1) // _BLK_ATT
    support = pl.pallas_call(
        _att_kernel,
        grid=(grid_att,),
        out_shape=jax.ShapeDtypeStruct((n, nclass), jnp.float32),
        in_specs=[
            pl.BlockSpec((_BLK_ATT, n), lambda i: (i, 0)),
            pl.BlockSpec((_BLK_ATT, n), lambda i: (i, 0)),
            pl.BlockSpec((_BLK_ATT, n), lambda i: (i, 0)),
            pl.BlockSpec((_BLK_ATT, n), lambda i: (i, 0)),
            pl.BlockSpec((n, hid2), lambda i: (0, 0)),
            pl.BlockSpec((_BLK_ATT, hid2), lambda i: (i, 0)),
            pl.BlockSpec((hid2, 1), lambda i: (0, 0)),
            pl.BlockSpec((hid2, 1), lambda i: (0, 0)),
            pl.BlockSpec((hid2, nclass), lambda i: (0, 0)),
        ],
        out_specs=pl.BlockSpec((_BLK_ATT, nclass), lambda i: (i, 0)),
        compiler_params=pltpu.CompilerParams(
            dimension_semantics=("parallel",),
        ),
    )(A_tilde, s1_sct, s2_sct, s3_sct, H, H, a0, a1, Wg)

    grid_fin = n // _BLK_FIN
    out = pl.pallas_call(
        _final_kernel,
        grid=(grid_fin,),
        out_shape=jax.ShapeDtypeStruct((n, nclass), jnp.float32),
        in_specs=[
            pl.BlockSpec((_BLK_FIN, n), lambda i: (i, 0)),
            pl.BlockSpec((n, nclass), lambda i: (0, 0)),
            pl.BlockSpec((1, nclass), lambda i: (0, 0)),
        ],
        out_specs=pl.BlockSpec((_BLK_FIN, nclass), lambda i: (i, 0)),
        compiler_params=pltpu.CompilerParams(
            dimension_semantics=("parallel",),
        ),
    )(A_tilde, support, bg2)
    return out
